# baseline (speedup 1.0000x reference)
import functools

import jax
import jax.numpy as jnp
from jax import lax
from jax.experimental import pallas as pl
from jax.experimental.pallas import tpu as pltpu
from jax.experimental.pallas import tpu_sc as plsc

NC = 2
NS = 16
TR = 2048
KD = 1000
KP = 400
GRAPHS = 64

_SC_MESH = plsc.VectorSubcoreMesh(core_axis_name="c", subcore_axis_name="s")
_SC_PARAMS = pltpu.CompilerParams(use_tc_tiling_on_sc=False)


def _dot(a, b):
  return lax.dot_general(a, b, (((1,), (0,)), ((), ())),
                         preferred_element_type=jnp.float32)


def _make_deg(E, NP):
  ept = E // (NC * NS)
  nchunk = ept // KD
  rows_pt = NP // NS

  @functools.partial(
      pl.kernel,
      out_type=[jax.ShapeDtypeStruct((NP,), jnp.float32)] * 2,
      mesh=_SC_MESH,
      compiler_params=_SC_PARAMS,
      scratch_types=[
          pltpu.VMEM((KD,), jnp.int32),
          pltpu.VMEM((KD,), jnp.float32),
          pltpu.VMEM_SHARED((NP,), jnp.float32),
      ],
  )
  def deg_kernel(dst_hbm, ones_hbm, zeros_hbm, deg0_hbm, deg1_hbm,
                 idx_v, ones_v, acc_s):
    cid = lax.axis_index("c")
    sid = lax.axis_index("s")
    pltpu.sync_copy(zeros_hbm, acc_s.at[pl.ds(sid * rows_pt, rows_pt)])
    pltpu.sync_copy(ones_hbm, ones_v)
    plsc.subcore_barrier()
    base = (cid * NS + sid) * ept

    def chunk(k, carry):
      pltpu.sync_copy(dst_hbm.at[pl.ds(base + k * KD, KD)], idx_v)
      pltpu.sync_copy(ones_v, acc_s.at[idx_v], add=True)
      return carry

    lax.fori_loop(0, nchunk, chunk, 0)
    plsc.subcore_barrier()

    @pl.when(cid == 0)
    def _():
      pltpu.sync_copy(acc_s.at[pl.ds(sid * rows_pt, rows_pt)],
                      deg0_hbm.at[pl.ds(sid * rows_pt, rows_pt)])

    @pl.when(cid == 1)
    def _():
      pltpu.sync_copy(acc_s.at[pl.ds(sid * rows_pt, rows_pt)],
                      deg1_hbm.at[pl.ds(sid * rows_pt, rows_pt)])

  return deg_kernel


def _make_prop(E, NP, nplanes):
  ept = E // NS
  nchunk = ept // KP
  rows_pt = NP // NS
  planes_pc = nplanes // NC
  assert nchunk % 2 == 1

  @functools.partial(
      pl.kernel,
      out_type=jax.ShapeDtypeStruct((nplanes, NP, 32), jnp.float32),
      mesh=_SC_MESH,
      compiler_params=_SC_PARAMS,
      scratch_types=[
          pltpu.VMEM((KP,), jnp.int32),
          pltpu.VMEM((KP,), jnp.int32),
          pltpu.VMEM((KP,), jnp.int32),
          pltpu.VMEM((KP,), jnp.int32),
          pltpu.VMEM((KP, 32), jnp.float32),
          pltpu.VMEM((KP, 32), jnp.float32),
          pltpu.VMEM_SHARED((NP, 32), jnp.float32),
          pltpu.SemaphoreType.DMA,
          pltpu.SemaphoreType.DMA,
          pltpu.SemaphoreType.DMA,
          pltpu.SemaphoreType.DMA,
      ],
  )
  def prop_kernel(htp_hbm, src_hbm, dst_hbm, zeros2_hbm, sp_hbm,
                  src_a, dst_a, src_b, dst_b,
                  rows_a, rows_b, acc_s, gsem_a, gsem_b, ssem_a, ssem_b):
    cid = lax.axis_index("c")
    sid = lax.axis_index("s")
    base = sid * ept
    row0 = sid * rows_pt

    def load_idx(k, sv, dv):
      off = base + k * KP
      pltpu.sync_copy(src_hbm.at[pl.ds(off, KP)], sv)
      pltpu.sync_copy(dst_hbm.at[pl.ds(off, KP)], dv)

    def one_plane(p):
      ht_hbm = htp_hbm.at[p]
      pltpu.sync_copy(zeros2_hbm, acc_s.at[pl.ds(row0, rows_pt)])
      plsc.subcore_barrier()

      def pair(i, carry):
        k = 2 * i
        load_idx(k, src_a, dst_a)
        pltpu.async_copy(ht_hbm.at[src_a], rows_a, gsem_a)
        load_idx(k + 1, src_b, dst_b)
        pltpu.async_copy(ht_hbm.at[src_b], rows_b, gsem_b)
        pltpu.make_async_copy(ht_hbm.at[src_a], rows_a, gsem_a).wait()
        pltpu.async_copy(rows_a, acc_s.at[dst_a], ssem_a, add=True)
        pltpu.make_async_copy(ht_hbm.at[src_b], rows_b, gsem_b).wait()
        pltpu.async_copy(rows_b, acc_s.at[dst_b], ssem_b, add=True)
        pltpu.make_async_copy(rows_a, acc_s.at[dst_a], ssem_a).wait()
        pltpu.make_async_copy(rows_b, acc_s.at[dst_b], ssem_b).wait()
        return carry

      lax.fori_loop(0, nchunk // 2, pair, 0)
      load_idx(nchunk - 1, src_a, dst_a)
      pltpu.async_copy(ht_hbm.at[src_a], rows_a, gsem_a)
      pltpu.make_async_copy(ht_hbm.at[src_a], rows_a, gsem_a).wait()
      pltpu.sync_copy(rows_a, acc_s.at[dst_a], add=True)
      plsc.subcore_barrier()
      pltpu.sync_copy(acc_s.at[pl.ds(row0, rows_pt)],
                      sp_hbm.at[p, pl.ds(row0, rows_pt)])
      plsc.subcore_barrier()

    @pl.when(cid == 0)
    def _():
      for f in range(planes_pc):
        one_plane(f)

    @pl.when(cid == 1)
    def _():
      for f in range(planes_pc):
        one_plane(planes_pc + f)

  return prop_kernel


def _mlp_body(xT_ref, W1_ref, b1_ref, W2_ref, b2_ref, W3_ref, b3_ref,
              Wc1_ref, d0_ref, d1_ref, htp_ref, dis_ref):
  h = lax.dot_general(xT_ref[...], W1_ref[...], (((0,), (0,)), ((), ())),
                      preferred_element_type=jnp.float32)
  h = jnp.maximum(h + b1_ref[...], 0.0)
  h = jnp.maximum(_dot(h, W2_ref[...]) + b2_ref[...], 0.0)
  h = jnp.maximum(_dot(h, W3_ref[...]) + b3_ref[...], 0.0)
  p1 = _dot(h, Wc1_ref[...])
  deg = d0_ref[...] + d1_ref[...] + 1.0
  dis = 1.0 / jnp.sqrt(deg)
  ht = p1 * dis
  htp_ref[0] = ht[:, :32]
  htp_ref[1] = ht[:, 32:]
  dis_ref[...] = dis


def _mid_body(sp_ref, htp_ref, dis_ref, bc1_ref, Wc2_ref, ht2p_ref):
  dis = dis_ref[...]
  h4 = jnp.concatenate(
      [jnp.maximum(dis * (sp_ref[p] + htp_ref[p])
                   + bc1_ref[:, 32 * p:32 * (p + 1)], 0.0) for p in range(2)],
      axis=1)
  hw2 = _dot(h4, Wc2_ref[...])
  ht2 = hw2 * dis
  for p in range(8):
    ht2p_ref[p] = ht2[:, 32 * p:32 * (p + 1)]


def _make_segmax(N):
  def body(lo_ref, hi_ref, s2p_ref, ht2p_ref, dis_ref, batch_ref, bc2_ref,
           out_ref):
    i = pl.program_id(0)

    @pl.when(i == 0)
    def _():
      out_ref[...] = jnp.full(out_ref.shape, -jnp.inf, jnp.float32)

    dis = dis_ref[...]
    h5 = jnp.concatenate(
        [dis * (s2p_ref[p] + ht2p_ref[p]) for p in range(8)], axis=1)
    h5 = jnp.maximum(h5 + bc2_ref[...], 0.0)
    bv = batch_ref[0]
    rowid = lax.broadcasted_iota(jnp.int32, (TR, 1), 0) + i * TR
    valid = rowid < N
    lo = lo_ref[i]
    hi = hi_ref[i]

    def gbody(g, carry):
      m = (bv == g) & valid
      cur = jnp.max(jnp.where(m, h5, -jnp.inf), axis=0, keepdims=True)
      out_ref[pl.ds(g, 1), :] = jnp.maximum(out_ref[pl.ds(g, 1), :], cur)
      return carry

    lax.fori_loop(lo, hi + 1, gbody, 0)

  return body


def _head_body(g_ref, Wl_ref, bl_ref, Wl2_ref, bl2_ref, Wl3_ref, bl3_ref,
               Wl4_ref, bl4_ref, out_ref):
  g = jnp.maximum(_dot(g_ref[...], Wl_ref[...]) + bl_ref[...], 0.0)
  g = jnp.maximum(_dot(g, Wl2_ref[...]) + bl2_ref[...], 0.0)
  g = jnp.maximum(_dot(g, Wl3_ref[...]) + bl3_ref[...], 0.0)
  out_ref[...] = _dot(g, Wl4_ref[...]) + bl4_ref[...]


def _row_spec(cols):
  return pl.BlockSpec((TR, cols), lambda i: (i, 0))


def _const_spec(shape):
  return pl.BlockSpec(shape, lambda i: tuple(0 for _ in shape))


def kernel(x, edge_index, batch, dropout, W1, b1, W2, b2, W3, b3, Wc1, bc1,
           Wc2, bc2, Wl, bl, Wl2, bl2, Wl3, bl3, Wl4, bl4):
  N = x.shape[0]
  E = edge_index.shape[1]
  NT = -(-N // TR)
  NP = NT * TR
  rows_pt = NP // NS

  src = edge_index[0]
  dst = edge_index[1]
  xT = jnp.pad(x.T, ((0, 0), (0, NP - N)))
  batch_p = jnp.pad(batch, (0, NP - N), constant_values=GRAPHS - 1)
  batch3 = batch_p.reshape(NT, TR, 1)
  tile_lo = batch_p.reshape(NT, TR)[:, 0]
  tile_hi = batch_p.reshape(NT, TR)[:, -1]

  ones_kd = jnp.ones((KD,), jnp.float32)
  zeros_1d = jnp.zeros((rows_pt,), jnp.float32)
  zeros_2d = jnp.zeros((rows_pt, 32), jnp.float32)

  deg0, deg1 = _make_deg(E, NP)(dst, ones_kd, zeros_1d)
  deg0 = deg0.reshape(NP, 1)
  deg1 = deg1.reshape(NP, 1)

  htp, dis = pl.pallas_call(
      _mlp_body,
      grid=(NT,),
      in_specs=[
          pl.BlockSpec((2, TR), lambda i: (0, i)),
          _const_spec((2, 64)), _const_spec((1, 64)),
          _const_spec((64, 128)), _const_spec((1, 128)),
          _const_spec((128, 512)), _const_spec((1, 512)),
          _const_spec((512, 64)),
          _row_spec(1), _row_spec(1),
      ],
      out_specs=[pl.BlockSpec((2, TR, 32), lambda i: (0, i, 0)),
                 _row_spec(1)],
      out_shape=[
          jax.ShapeDtypeStruct((2, NP, 32), jnp.float32),
          jax.ShapeDtypeStruct((NP, 1), jnp.float32),
      ],
  )(xT, W1, b1.reshape(1, 64), W2, b2.reshape(1, 128),
    W3, b3.reshape(1, 512), Wc1, deg0, deg1)

  sp = _make_prop(E, NP, 2)(htp, src, dst, zeros_2d)

  ht2p = pl.pallas_call(
      _mid_body,
      grid=(NT,),
      in_specs=[pl.BlockSpec((2, TR, 32), lambda i: (0, i, 0)),
                pl.BlockSpec((2, TR, 32), lambda i: (0, i, 0)),
                _row_spec(1), _const_spec((1, 64)), _const_spec((64, 256))],
      out_specs=pl.BlockSpec((8, TR, 32), lambda i: (0, i, 0)),
      out_shape=jax.ShapeDtypeStruct((8, NP, 32), jnp.float32),
  )(sp, htp, dis, bc1.reshape(1, 64), Wc2)

  s2p = _make_prop(E, NP, 8)(ht2p, src, dst, zeros_2d)

  gmax = pl.pallas_call(
      _make_segmax(N),
      grid_spec=pltpu.PrefetchScalarGridSpec(
          num_scalar_prefetch=2,
          grid=(NT,),
          in_specs=[
              pl.BlockSpec((8, TR, 32), lambda i, lo, hi: (0, i, 0)),
              pl.BlockSpec((8, TR, 32), lambda i, lo, hi: (0, i, 0)),
              pl.BlockSpec((TR, 1), lambda i, lo, hi: (i, 0)),
              pl.BlockSpec((1, TR, 1), lambda i, lo, hi: (i, 0, 0)),
              pl.BlockSpec((1, 256), lambda i, lo, hi: (0, 0)),
          ],
          out_specs=pl.BlockSpec((GRAPHS, 256), lambda i, lo, hi: (0, 0)),
      ),
      out_shape=jax.ShapeDtypeStruct((GRAPHS, 256), jnp.float32),
  )(tile_lo, tile_hi, s2p, ht2p, dis, batch3, bc2.reshape(1, 256))

  out = pl.pallas_call(
      _head_body,
      in_specs=[pl.BlockSpec((GRAPHS, 256), lambda: (0, 0))] + [
          pl.BlockSpec(s, lambda: (0, 0)) for s in
          [(256, 128), (1, 128), (128, 64), (1, 64),
           (64, 28), (1, 28), (28, 2), (1, 2)]
      ],
      out_specs=pl.BlockSpec((GRAPHS, 2), lambda: (0, 0)),
      out_shape=jax.ShapeDtypeStruct((GRAPHS, 2), jnp.float32),
  )(gmax, Wl, bl.reshape(1, 128), Wl2, bl2.reshape(1, 64),
    Wl3, bl3.reshape(1, 28), Wl4, bl4.reshape(1, 2))

  return out
